# baseline (speedup 1.0000x reference)
import functools

import jax
import jax.numpy as jnp
from jax.experimental import pallas as pl
from jax.experimental.pallas import tpu as pltpu

_V = 16


def _fold_kernel(L, E, gt_ref, rt_ref, w_ref, b_ref, tt_ref, e_ref):
    blocks = []
    for j in range(2 * L):
        tab = gt_ref[:_V, :] if j < L else rt_ref[:_V, :]
        wblk = w_ref[j * E:(j + 1) * E, :]
        blocks.append(jax.lax.dot_general(
            wblk, tab, (((0,), (1,)), ((), ())),
            preferred_element_type=jnp.float32))
    blocks.append(b_ref[...])
    tt_ref[...] = jnp.concatenate(blocks, axis=1)
    C = 2 * L * _V
    cc = jax.lax.broadcasted_iota(jnp.int32, (C + 1, 2 * L + 1), 0)
    jj = jax.lax.broadcasted_iota(jnp.int32, (C + 1, 2 * L + 1), 1)
    sel = (jj == cc // _V).astype(jnp.int32)
    aug = jnp.where(jj == 2 * L, -(cc % _V), sel)
    e_ref[...] = jnp.where(cc == C, 0, aug).astype(jnp.bfloat16)


def _main_kernel(L, H, SB, Bd, g_ref, r_ref, tt_ref, e_ref, o_ref):
    g = g_ref[...]
    r = r_ref[...]
    ones = jnp.ones((1, SB * Bd), jnp.int32)
    idx = jnp.concatenate([g, r, ones], axis=0)
    idxb = idx.astype(jnp.bfloat16)
    d = jnp.dot(e_ref[...], idxb,
                preferred_element_type=jnp.float32)
    oh = jnp.where(d == 0, jnp.float32(1), jnp.float32(0))
    for s in range(SB):
        acc = jnp.dot(tt_ref[...], oh[:, s * Bd:(s + 1) * Bd],
                      preferred_element_type=jnp.float32)
        o_ref[s * H:(s + 1) * H, :] = acc


def kernel(goal, rules, goal_table, rules_table, W, b, training):
    Bd, Sd, L = goal.shape
    E = goal_table.shape[1]
    H = W.shape[1]
    C1 = 2 * L * _V + 1
    SB = 8
    assert Sd % SB == 0

    Tt, Emat = pl.pallas_call(
        functools.partial(_fold_kernel, L, E),
        out_shape=[
            jax.ShapeDtypeStruct((H, C1), jnp.float32),
            jax.ShapeDtypeStruct((C1, 2 * L + 1), jnp.bfloat16),
        ],
    )(goal_table, rules_table, W, b.reshape(H, 1))

    gT = jnp.transpose(goal, (2, 1, 0)).reshape(L, Sd * Bd)
    rT = jnp.transpose(rules, (2, 1, 0)).reshape(L, Sd * Bd)

    out2 = pl.pallas_call(
        functools.partial(_main_kernel, L, H, SB, Bd),
        grid=(Sd // SB,),
        in_specs=[
            pl.BlockSpec((L, SB * Bd), lambda i: (0, i)),
            pl.BlockSpec((L, SB * Bd), lambda i: (0, i)),
            pl.BlockSpec((H, C1), lambda i: (0, 0)),
            pl.BlockSpec((C1, 2 * L + 1), lambda i: (0, 0)),
        ],
        out_specs=pl.BlockSpec((SB * H, Bd), lambda i: (i, 0)),
        out_shape=jax.ShapeDtypeStruct((Sd * H, Bd), jnp.float32),
        compiler_params=pltpu.CompilerParams(
            dimension_semantics=("arbitrary",),
        ),
    )(gT, rT, Tt, Emat)
    return jnp.transpose(out2.reshape(Sd, H, Bd), (2, 0, 1))


# speedup vs baseline: 1.4590x; 1.4590x over previous
import functools

import jax
import jax.numpy as jnp
from jax.experimental import pallas as pl
from jax.experimental.pallas import tpu as pltpu

_V = 16


def _fold_kernel(L, E, gt_ref, rt_ref, w_ref, b_ref, tt_ref, e_ref):
    blocks = []
    for j in range(2 * L):
        tab = gt_ref[:_V, :] if j < L else rt_ref[:_V, :]
        wblk = w_ref[j * E:(j + 1) * E, :]
        blocks.append(jax.lax.dot_general(
            wblk, tab, (((0,), (1,)), ((), ())),
            preferred_element_type=jnp.float32))
    blocks.append(b_ref[...])
    tt_ref[...] = jnp.concatenate(blocks, axis=1)
    C = 2 * L * _V
    cc = jax.lax.broadcasted_iota(jnp.int32, (C + 1, 2 * L + 1), 0)
    jj = jax.lax.broadcasted_iota(jnp.int32, (C + 1, 2 * L + 1), 1)
    sel = (jj == cc // _V).astype(jnp.int32)
    aug = jnp.where(jj == 2 * L, -(cc % _V), sel)
    e_ref[...] = jnp.where(cc == C, 0, aug).astype(jnp.bfloat16)


def _main_kernel(L, H, SB, Bd, g_ref, r_ref, tt_ref, e_ref, o_ref):
    g = g_ref[...].reshape(L, SB * Bd)
    r = r_ref[...].reshape(L, SB * Bd)
    ones = jnp.ones((1, SB * Bd), jnp.int32)
    idx = jnp.concatenate([g, r, ones], axis=0)
    idxb = idx.astype(jnp.bfloat16)
    d = jnp.dot(e_ref[...], idxb,
                preferred_element_type=jnp.float32)
    oh = jnp.where(d == 0, jnp.float32(1), jnp.float32(0))
    for s in range(SB):
        acc = jnp.dot(tt_ref[...], oh[:, s * Bd:(s + 1) * Bd],
                      preferred_element_type=jnp.float32)
        o_ref[s * H:(s + 1) * H, :] = acc


def kernel(goal, rules, goal_table, rules_table, W, b, training):
    Bd, Sd, L = goal.shape
    E = goal_table.shape[1]
    H = W.shape[1]
    C1 = 2 * L * _V + 1
    SB = 8
    assert Sd % SB == 0

    Tt, Emat = pl.pallas_call(
        functools.partial(_fold_kernel, L, E),
        out_shape=[
            jax.ShapeDtypeStruct((H, C1), jnp.float32),
            jax.ShapeDtypeStruct((C1, 2 * L + 1), jnp.bfloat16),
        ],
    )(goal_table, rules_table, W, b.reshape(H, 1))

    gT = jnp.transpose(goal, (2, 1, 0))
    rT = jnp.transpose(rules, (2, 1, 0))

    out2 = pl.pallas_call(
        functools.partial(_main_kernel, L, H, SB, Bd),
        grid=(Sd // SB,),
        in_specs=[
            pl.BlockSpec((L, SB, Bd), lambda i: (0, i, 0)),
            pl.BlockSpec((L, SB, Bd), lambda i: (0, i, 0)),
            pl.BlockSpec((H, C1), lambda i: (0, 0)),
            pl.BlockSpec((C1, 2 * L + 1), lambda i: (0, 0)),
        ],
        out_specs=pl.BlockSpec((SB * H, Bd), lambda i: (i, 0)),
        out_shape=jax.ShapeDtypeStruct((Sd * H, Bd), jnp.float32),
        compiler_params=pltpu.CompilerParams(
            dimension_semantics=("arbitrary",),
        ),
    )(gT, rT, Tt, Emat)
    return jnp.transpose(out2.reshape(Sd, H, Bd), (2, 0, 1))


# speedup vs baseline: 1.4907x; 1.0218x over previous
import functools

import jax
import jax.numpy as jnp
from jax.experimental import pallas as pl
from jax.experimental.pallas import tpu as pltpu

_V = 16


def _fold_kernel(L, E, gt_ref, rt_ref, w_ref, tt_ref, e_ref):
    blocks = []
    for j in range(2 * L):
        tab = gt_ref[:_V, :] if j < L else rt_ref[:_V, :]
        wblk = w_ref[j * E:(j + 1) * E, :]
        blocks.append(jax.lax.dot_general(
            wblk, tab, (((0,), (1,)), ((), ())),
            preferred_element_type=jnp.float32))
    tt_ref[...] = jnp.concatenate(blocks, axis=1)
    C = 2 * L * _V
    cc = jax.lax.broadcasted_iota(jnp.int32, (C, 2 * L + 1), 0)
    jj = jax.lax.broadcasted_iota(jnp.int32, (C, 2 * L + 1), 1)
    sel = (jj == cc // _V).astype(jnp.int32)
    aug = jnp.where(jj == 2 * L, -(cc % _V), sel)
    e_ref[...] = aug.astype(jnp.bfloat16)


def _main_kernel(L, H, SB, g_ref, r_ref, tt_ref, e_ref, b_ref, o_ref):
    Bd = g_ref.shape[2]
    g = g_ref[...].reshape(L, SB * Bd)
    r = r_ref[...].reshape(L, SB * Bd)
    ones = jnp.ones((1, SB * Bd), jnp.int32)
    idx = jnp.concatenate([g, r, ones], axis=0)
    idxb = idx.astype(jnp.bfloat16)
    d = jnp.dot(e_ref[...], idxb,
                preferred_element_type=jnp.float32)
    oh = jnp.where(d == 0.0, jnp.float32(1), jnp.float32(0))
    for s in range(SB):
        acc = jnp.dot(tt_ref[...], oh[:, s * Bd:(s + 1) * Bd],
                      preferred_element_type=jnp.float32)
        o_ref[s * H:(s + 1) * H, :] = acc + b_ref[...]


def kernel(goal, rules, goal_table, rules_table, W, b, training):
    Bd, Sd, L = goal.shape
    E = goal_table.shape[1]
    H = W.shape[1]
    SB = 8
    assert Sd % SB == 0

    Tt, Emat = pl.pallas_call(
        functools.partial(_fold_kernel, L, E),
        out_shape=[
            jax.ShapeDtypeStruct((H, 2 * L * _V), jnp.float32),
            jax.ShapeDtypeStruct((2 * L * _V, 2 * L + 1), jnp.bfloat16),
        ],
    )(goal_table, rules_table, W)

    gT = jnp.transpose(goal, (2, 1, 0))
    rT = jnp.transpose(rules, (2, 1, 0))

    out2 = pl.pallas_call(
        functools.partial(_main_kernel, L, H, SB),
        grid=(Sd // SB,),
        in_specs=[
            pl.BlockSpec((L, SB, Bd), lambda i: (0, i, 0)),
            pl.BlockSpec((L, SB, Bd), lambda i: (0, i, 0)),
            pl.BlockSpec((H, 2 * L * _V), lambda i: (0, 0)),
            pl.BlockSpec((2 * L * _V, 2 * L + 1), lambda i: (0, 0)),
            pl.BlockSpec((H, 1), lambda i: (0, 0)),
        ],
        out_specs=pl.BlockSpec((SB * H, Bd), lambda i: (i, 0)),
        out_shape=jax.ShapeDtypeStruct((Sd * H, Bd), jnp.float32),
        compiler_params=pltpu.CompilerParams(
            dimension_semantics=("arbitrary",),
        ),
    )(gT, rT, Tt, Emat, b.reshape(H, 1))
    return jnp.transpose(out2.reshape(Sd, H, Bd), (2, 0, 1))
